# Initial kernel scaffold; baseline (speedup 1.0000x reference)
import jax
import jax.numpy as jnp
from jax.experimental import pallas as pl


def kernel(h, edge_index, edge_dist, pW0, pb0, pg0, pbe0, pW1, pb1, pg1, pbe1, l0W, l0b, l0g, l0be, eW1, eb1, eW2, eb2):
    raise NotImplementedError("write your pallas kernel here")


# speedup vs baseline: 1.4015x; 1.4015x over previous
import functools

import jax
import jax.numpy as jnp
from jax import lax
from jax.experimental import pallas as pl
from jax.experimental.pallas import tpu as pltpu
from jax.experimental.pallas import tpu_sc as plsc

N = 50000
E = 800000
NC = 2
NS = 16
L = 16

E_PAD = 819200
HALF = 25088
RPS = HALF // NS
SH_ROWS = HALF + 8
CH = 80
B2 = 160
NCH2 = B2 // CH
EPS2 = E_PAD // NS
NBLK2 = EPS2 // B2
BD = 1600
NCHD = BD // CH
NBLKD = EPS2 // BD
B4 = 400
NCH4 = B4 // CH
EPW4 = E_PAD // (NC * NS)
NBLK4 = EPW4 // B4

_HIGHEST = jax.lax.Precision.HIGHEST
_SC_PARAMS = pltpu.CompilerParams(needs_layout_passes=False,
                                  use_tc_tiling_on_sc=False)


def _dot(a, b):
    return jax.lax.dot_general(a, b, (((1,), (0,)), ((), ())),
                               preferred_element_type=jnp.float32,
                               precision=_HIGHEST)


def _ln(x, g, b, eps=1e-5):
    mu = jnp.mean(x, axis=-1, keepdims=True)
    var = jnp.mean((x - mu) ** 2, axis=-1, keepdims=True)
    return (x - mu) / jnp.sqrt(var + eps) * g + b


def _proj_body(h_ref, w0, b0, g0, e0, w1, b1, g1, e1, o_ref):
    h = h_ref[...]

    def chunk(x, W, b, g, be):
        y = _dot(x, W[...]) + b[...]
        y = _ln(y, g[...], be[...])
        return jnp.where(y >= 0, y, 0.01 * y)

    p0 = chunk(h[:, :64], w0, b0, g0, e0)
    p1 = chunk(h[:, 64:], w1, b1, g1, e1)
    o_ref[...] = jnp.concatenate([p0, p1], axis=1)


def _proj(h, pW0, pb0, pg0, pbe0, pW1, pb1, pg1, pbe1):
    blk = 1000
    full = lambda shape: pl.BlockSpec(shape, lambda i: (0, 0))
    return pl.pallas_call(
        _proj_body,
        grid=(N // blk,),
        in_specs=[pl.BlockSpec((blk, 128), lambda i: (i, 0)),
                  full((64, 32)), full((1, 32)), full((1, 32)), full((1, 32)),
                  full((64, 32)), full((1, 32)), full((1, 32)), full((1, 32))],
        out_specs=pl.BlockSpec((blk, 64), lambda i: (i, 0)),
        out_shape=jax.ShapeDtypeStruct((N, 64), jnp.float32),
    )(h, pW0, pb0.reshape(1, 32), pg0.reshape(1, 32), pbe0.reshape(1, 32),
      pW1, pb1.reshape(1, 32), pg1.reshape(1, 32), pbe1.reshape(1, 32))


def _sc_degree(dst1d):
    mesh = plsc.VectorSubcoreMesh(core_axis_name="c", subcore_axis_name="s")

    @functools.partial(
        pl.kernel,
        out_type=jax.ShapeDtypeStruct((2 * HALF, L), jnp.float32),
        mesh=mesh,
        scratch_types=[
            pltpu.VMEM_SHARED((SH_ROWS, L), jnp.float32),
            pltpu.VMEM((BD,), jnp.int32),
            pltpu.VMEM((NCHD, CH), jnp.int32),
            pltpu.VMEM((BD, L), jnp.float32),
            pltpu.VMEM((32, L), jnp.float32),
            pltpu.SemaphoreType.DMA,
        ],
        compiler_params=_SC_PARAMS,
    )
    def deg(dst_hbm, out_hbm, shared, dbuf, sidx, ones, zb, sem_s):
        c = lax.axis_index("c")
        s = lax.axis_index("s")
        lo = c * HALF

        zero16 = jnp.zeros((L,), jnp.float32)
        onev = (lax.iota(jnp.int32, L) == 0).astype(jnp.float32)

        @pl.loop(0, 32)
        def _(r):
            zb[r, pl.ds(0, L)] = zero16

        @pl.loop(0, BD)
        def _(r):
            ones[r, pl.ds(0, L)] = onev

        for kk in range(49):
            pltpu.sync_copy(zb, shared.at[pl.ds(s * RPS + kk * 32, 32)])

        @pl.when(s == 0)
        def _():
            pltpu.sync_copy(zb.at[pl.ds(0, 8)], shared.at[pl.ds(HALF, 8)])

        plsc.subcore_barrier()

        @pl.loop(0, NBLKD)
        def _(i):
            flat0 = s * EPS2 + i * BD
            pltpu.sync_copy(dst_hbm.at[pl.ds(flat0, BD)], dbuf)
            for p in range(BD // L):
                dd = dbuf[pl.ds(p * L, L)]
                loc = dd - lo
                inr = (loc >= 0) & (loc < HALF)
                si = jnp.where(inr, loc, HALF)
                sidx[p // (CH // L), pl.ds((p % (CH // L)) * L, L)] = si
            scs = [pltpu.async_copy(ones.at[pl.ds(j * CH, CH)],
                                    shared.at[sidx.at[j]], sem_s, add=True)
                   for j in range(NCHD)]
            for h_ in scs:
                h_.wait()

        plsc.subcore_barrier()
        pltpu.sync_copy(shared.at[pl.ds(s * RPS, RPS)],
                        out_hbm.at[pl.ds(c * HALF + s * RPS, RPS)])

    return deg(dst1d)


def _sc_aggregate(hp, src1d, dst1d, dist1d):
    mesh = plsc.VectorSubcoreMesh(core_axis_name="c", subcore_axis_name="s")

    @functools.partial(
        pl.kernel,
        out_type=jax.ShapeDtypeStruct((2 * HALF, 64), jnp.float32),
        mesh=mesh,
        scratch_types=[
            pltpu.VMEM_SHARED((SH_ROWS, 64), jnp.float32),
            pltpu.VMEM((B2,), jnp.int32),
            pltpu.VMEM((NCH2, CH), jnp.int32),
            pltpu.VMEM((B2,), jnp.int32),
            pltpu.VMEM((B2,), jnp.float32),
            pltpu.VMEM((B2, 64), jnp.float32),
            pltpu.VMEM((B2, 64), jnp.float32),
            pltpu.VMEM((32, 64), jnp.float32),
            pltpu.SemaphoreType.DMA,
            pltpu.SemaphoreType.DMA,
        ],
        compiler_params=_SC_PARAMS,
    )
    def agg(hp_hbm, src_hbm, dst_hbm, dist_hbm, out_hbm,
            shared, gidx, sidx, dbuf, distb, rows, m, zb, sem_g, sem_s):
        c = lax.axis_index("c")
        s = lax.axis_index("s")
        lo = c * HALF

        zero16 = jnp.zeros((L,), jnp.float32)

        @pl.loop(0, 32)
        def _(r):
            for q in range(4):
                zb[r, pl.ds(q * L, L)] = zero16

        for kk in range(49):
            pltpu.sync_copy(zb, shared.at[pl.ds(s * RPS + kk * 32, 32)])

        @pl.when(s == 0)
        def _():
            pltpu.sync_copy(zb.at[pl.ds(0, 8)], shared.at[pl.ds(HALF, 8)])

        plsc.subcore_barrier()

        @pl.loop(0, NBLK2)
        def _(i):
            flat0 = s * EPS2 + i * B2
            pltpu.sync_copy(src_hbm.at[pl.ds(flat0, B2)], gidx)
            pltpu.sync_copy(dst_hbm.at[pl.ds(flat0, B2)], dbuf)
            pltpu.sync_copy(dist_hbm.at[pl.ds(flat0, B2)], distb)
            cps = [pltpu.async_copy(hp_hbm.at[gidx.at[pl.ds(j * CH, CH)]],
                                    rows.at[pl.ds(j * CH, CH)], sem_g)
                   for j in range(NCH2)]
            for p in range(B2 // L):
                dd = dbuf[pl.ds(p * L, L)]
                loc = dd - lo
                inr = (loc >= 0) & (loc < HALF)
                si = jnp.where(inr, loc, HALF)
                sidx[p // (CH // L), pl.ds((p % (CH // L)) * L, L)] = si
            for h_ in cps:
                h_.wait()

            @pl.loop(0, B2)
            def _(r):
                d = plsc.load_gather(distb, [jnp.full((L,), r, jnp.int32)])
                for q in range(4):
                    m[r, pl.ds(q * L, L)] = rows[r, pl.ds(q * L, L)] * d

            scs = [pltpu.async_copy(m.at[pl.ds(j * CH, CH)],
                                    shared.at[sidx.at[j]], sem_s, add=True)
                   for j in range(NCH2)]
            for h_ in scs:
                h_.wait()

        plsc.subcore_barrier()
        pltpu.sync_copy(shared.at[pl.ds(s * RPS, RPS)],
                        out_hbm.at[pl.ds(c * HALF + s * RPS, RPS)])

    return agg(hp, src1d, dst1d, dist1d)


def _node_body(hp_ref, ah_ref, deg_ref, w_ref, b_ref, g_ref, e_ref, o_ref):
    hp = hp_ref[...]
    ah = ah_ref[...]
    deg = deg_ref[:, 0:1]
    norm = jnp.where(deg > 0, 1.0 / jnp.maximum(deg, 1.0), 0.0)
    w = w_ref[...]
    y = _dot(hp, w[:64]) + _dot(ah * norm, w[64:]) + b_ref[...]
    y = _ln(y, g_ref[...], e_ref[...])
    o_ref[...] = jnp.maximum(y, 0.0)


def _node_mlp(hp, ahf, degf, l0W, l0b, l0g, l0be):
    blk = 1000
    full = lambda shape: pl.BlockSpec(shape, lambda i: (0, 0))
    return pl.pallas_call(
        _node_body,
        grid=(N // blk,),
        in_specs=[pl.BlockSpec((blk, 64), lambda i: (i, 0)),
                  pl.BlockSpec((blk, 64), lambda i: (i, 0)),
                  pl.BlockSpec((blk, L), lambda i: (i, 0)),
                  full((128, 64)), full((1, 64)), full((1, 64)), full((1, 64))],
        out_specs=pl.BlockSpec((blk, 64), lambda i: (i, 0)),
        out_shape=jax.ShapeDtypeStruct((N, 64), jnp.float32),
    )(hp, ahf, degf, l0W, l0b.reshape(1, 64), l0g.reshape(1, 64),
      l0be.reshape(1, 64))


def _sc_edge_product(hl, src1d, dst1d):
    mesh = plsc.VectorSubcoreMesh(core_axis_name="c", subcore_axis_name="s")

    @functools.partial(
        pl.kernel,
        out_type=jax.ShapeDtypeStruct((E_PAD, 64), jnp.float32),
        mesh=mesh,
        scratch_types=[
            pltpu.VMEM((B4,), jnp.int32),
            pltpu.VMEM((B4,), jnp.int32),
            pltpu.VMEM((B4, 64), jnp.float32),
            pltpu.VMEM((B4, 64), jnp.float32),
            pltpu.SemaphoreType.DMA,
        ],
        compiler_params=_SC_PARAMS,
    )
    def prod(hl_hbm, src_hbm, dst_hbm, t_hbm, ui, vi, hu, hv, sem):
        c = lax.axis_index("c")
        s = lax.axis_index("s")
        w = s * NC + c

        @pl.loop(0, NBLK4)
        def _(i):
            flat0 = w * EPW4 + i * B4
            pltpu.sync_copy(src_hbm.at[pl.ds(flat0, B4)], ui)
            pltpu.sync_copy(dst_hbm.at[pl.ds(flat0, B4)], vi)
            cps = [pltpu.async_copy(hl_hbm.at[ui.at[pl.ds(j * CH, CH)]],
                                    hu.at[pl.ds(j * CH, CH)], sem)
                   for j in range(NCH4)]
            cps += [pltpu.async_copy(hl_hbm.at[vi.at[pl.ds(j * CH, CH)]],
                                     hv.at[pl.ds(j * CH, CH)], sem)
                    for j in range(NCH4)]
            for h_ in cps:
                h_.wait()

            @pl.loop(0, B4)
            def _(r):
                for q in range(4):
                    hu[r, pl.ds(q * L, L)] = (hu[r, pl.ds(q * L, L)]
                                              * hv[r, pl.ds(q * L, L)])

            pltpu.sync_copy(hu, t_hbm.at[pl.ds(flat0, B4)])

    return prod(hl, src1d, dst1d)


def _edge_body(t_ref, w1, b1, w2, b2, o_ref):
    y = jnp.maximum(_dot(t_ref[...], w1[...]) + b1[...], 0.0)
    o_ref[...] = _dot(y, w2[...]) + b2[...]


def _edge_mlp(t, eW1, eb1, eW2, eb2):
    blk = 3200
    full = lambda shape: pl.BlockSpec(shape, lambda i: (0, 0))
    return pl.pallas_call(
        _edge_body,
        grid=(E_PAD // blk,),
        in_specs=[pl.BlockSpec((blk, 64), lambda i: (i, 0)),
                  full((64, 32)), full((1, 32)), full((32, 2)), full((1, 2))],
        out_specs=pl.BlockSpec((blk, 2), lambda i: (i, 0)),
        out_shape=jax.ShapeDtypeStruct((E_PAD, 2), jnp.float32),
    )(t, eW1, eb1.reshape(1, 32), eW2, eb2.reshape(1, 2))


def kernel(h, edge_index, edge_dist, pW0, pb0, pg0, pbe0, pW1, pb1, pg1, pbe1,
           l0W, l0b, l0g, l0be, eW1, eb1, eW2, eb2):
    src = edge_index[0]
    dst = edge_index[1]
    npad = E_PAD - E
    src_p = jnp.concatenate([src, jnp.zeros((npad,), jnp.int32)])
    dst_p2 = jnp.concatenate([dst, jnp.full((npad,), N, jnp.int32)])
    dst_p4 = jnp.concatenate([dst, jnp.zeros((npad,), jnp.int32)])
    dist_p = jnp.concatenate([edge_dist, jnp.zeros((npad,), jnp.float32)])

    degf = _sc_degree(dst_p2)
    hp = _proj(h, pW0, pb0, pg0, pbe0, pW1, pb1, pg1, pbe1)
    ahf = _sc_aggregate(hp, src_p, dst_p2, dist_p)
    hl = _node_mlp(hp, ahf[:N], degf[:N], l0W, l0b, l0g, l0be)
    t = _sc_edge_product(hl, src_p, dst_p4)
    score = _edge_mlp(t, eW1, eb1, eW2, eb2)
    return score[:E]


# speedup vs baseline: 1.8193x; 1.2980x over previous
import functools

import jax
import jax.numpy as jnp
from jax import lax
from jax.experimental import pallas as pl
from jax.experimental.pallas import tpu as pltpu
from jax.experimental.pallas import tpu_sc as plsc

N = 50000
E = 800000
NC = 2
NS = 16
L = 16

E_PAD = 819200
HALF = 25088
RPS = HALF // NS
SH_ROWS = HALF + 8
CH = 80
B2 = 160
NCH2 = B2 // CH
EPS2 = E_PAD // NS
NBLK2 = EPS2 // B2
BD = 1600
NCHD = BD // CH
NBLKD = EPS2 // BD
B4 = 400
NCH4 = B4 // CH
EPW4 = E_PAD // (NC * NS)
NBLK4 = EPW4 // B4

_HIGHEST = jax.lax.Precision.HIGHEST
_SC_PARAMS = pltpu.CompilerParams(needs_layout_passes=False,
                                  use_tc_tiling_on_sc=False)


def _dot(a, b):
    return jax.lax.dot_general(a, b, (((1,), (0,)), ((), ())),
                               preferred_element_type=jnp.float32,
                               precision=_HIGHEST)


def _ln(x, g, b, eps=1e-5):
    mu = jnp.mean(x, axis=-1, keepdims=True)
    var = jnp.mean((x - mu) ** 2, axis=-1, keepdims=True)
    return (x - mu) / jnp.sqrt(var + eps) * g + b


def _proj_body(h_ref, w0, b0, g0, e0, w1, b1, g1, e1, o_ref):
    h = h_ref[...]

    def chunk(x, W, b, g, be):
        y = _dot(x, W[...]) + b[...]
        y = _ln(y, g[...], be[...])
        return jnp.where(y >= 0, y, 0.01 * y)

    p0 = chunk(h[:, :64], w0, b0, g0, e0)
    p1 = chunk(h[:, 64:], w1, b1, g1, e1)
    o_ref[...] = jnp.concatenate([p0, p1], axis=1)


def _proj(h, pW0, pb0, pg0, pbe0, pW1, pb1, pg1, pbe1):
    blk = 1000
    full = lambda shape: pl.BlockSpec(shape, lambda i: (0, 0))
    return pl.pallas_call(
        _proj_body,
        grid=(N // blk,),
        in_specs=[pl.BlockSpec((blk, 128), lambda i: (i, 0)),
                  full((64, 32)), full((1, 32)), full((1, 32)), full((1, 32)),
                  full((64, 32)), full((1, 32)), full((1, 32)), full((1, 32))],
        out_specs=pl.BlockSpec((blk, 64), lambda i: (i, 0)),
        out_shape=jax.ShapeDtypeStruct((N, 64), jnp.float32),
    )(h, pW0, pb0.reshape(1, 32), pg0.reshape(1, 32), pbe0.reshape(1, 32),
      pW1, pb1.reshape(1, 32), pg1.reshape(1, 32), pbe1.reshape(1, 32))


def _sc_degree(dst1d):
    mesh = plsc.VectorSubcoreMesh(core_axis_name="c", subcore_axis_name="s")

    @functools.partial(
        pl.kernel,
        out_type=jax.ShapeDtypeStruct((2 * HALF, L), jnp.float32),
        mesh=mesh,
        scratch_types=[
            pltpu.VMEM_SHARED((SH_ROWS, L), jnp.float32),
            pltpu.VMEM((BD,), jnp.int32),
            pltpu.VMEM((NCHD, CH), jnp.int32),
            pltpu.VMEM((BD, L), jnp.float32),
            pltpu.VMEM((32, L), jnp.float32),
            pltpu.SemaphoreType.DMA,
        ],
        compiler_params=_SC_PARAMS,
    )
    def deg(dst_hbm, out_hbm, shared, dbuf, sidx, ones, zb, sem_s):
        c = lax.axis_index("c")
        s = lax.axis_index("s")
        lo = c * HALF

        zero16 = jnp.zeros((L,), jnp.float32)
        onev = (lax.iota(jnp.int32, L) == 0).astype(jnp.float32)

        @pl.loop(0, 32)
        def _(r):
            zb[r, pl.ds(0, L)] = zero16

        @pl.loop(0, BD)
        def _(r):
            ones[r, pl.ds(0, L)] = onev

        for kk in range(49):
            pltpu.sync_copy(zb, shared.at[pl.ds(s * RPS + kk * 32, 32)])

        @pl.when(s == 0)
        def _():
            pltpu.sync_copy(zb.at[pl.ds(0, 8)], shared.at[pl.ds(HALF, 8)])

        plsc.subcore_barrier()

        @pl.loop(0, NBLKD)
        def _(i):
            flat0 = s * EPS2 + i * BD
            pltpu.sync_copy(dst_hbm.at[pl.ds(flat0, BD)], dbuf)
            for p in range(BD // L):
                dd = dbuf[pl.ds(p * L, L)]
                loc = dd - lo
                inr = (loc >= 0) & (loc < HALF)
                si = jnp.where(inr, loc, HALF)
                sidx[p // (CH // L), pl.ds((p % (CH // L)) * L, L)] = si
            scs = [pltpu.async_copy(ones.at[pl.ds(j * CH, CH)],
                                    shared.at[sidx.at[j]], sem_s, add=True)
                   for j in range(NCHD)]
            for h_ in scs:
                h_.wait()

        plsc.subcore_barrier()
        pltpu.sync_copy(shared.at[pl.ds(s * RPS, RPS)],
                        out_hbm.at[pl.ds(c * HALF + s * RPS, RPS)])

    return deg(dst1d)


def _sc_aggregate(hp, src1d, dst1d, dist1d):
    mesh = plsc.VectorSubcoreMesh(core_axis_name="c", subcore_axis_name="s")

    @functools.partial(
        pl.kernel,
        out_type=jax.ShapeDtypeStruct((2 * HALF, 64), jnp.float32),
        mesh=mesh,
        scratch_types=[
            pltpu.VMEM_SHARED((SH_ROWS, 64), jnp.float32),
            pltpu.VMEM((2, B2), jnp.int32),
            pltpu.VMEM((2, NCH2, CH), jnp.int32),
            pltpu.VMEM((2, B2), jnp.int32),
            pltpu.VMEM((2, B2), jnp.float32),
            pltpu.VMEM((2, B2, 64), jnp.float32),
            pltpu.VMEM((32, 64), jnp.float32),
            pltpu.SemaphoreType.DMA, pltpu.SemaphoreType.DMA,
            pltpu.SemaphoreType.DMA, pltpu.SemaphoreType.DMA,
            pltpu.SemaphoreType.DMA, pltpu.SemaphoreType.DMA,
        ],
        compiler_params=_SC_PARAMS,
    )
    def agg(hp_hbm, src_hbm, dst_hbm, dist_hbm, out_hbm,
            shared, gidx, sidx, dbuf, distb, rows, zb,
            si0, si1, sg0, sg1, ss0, ss1):
        c = lax.axis_index("c")
        s = lax.axis_index("s")
        lo = c * HALF
        sem_i = [si0, si1]
        sem_g = [sg0, sg1]
        sem_s = [ss0, ss1]

        zero16 = jnp.zeros((L,), jnp.float32)

        @pl.loop(0, 32)
        def _(r):
            for q in range(4):
                zb[r, pl.ds(q * L, L)] = zero16

        for kk in range(49):
            pltpu.sync_copy(zb, shared.at[pl.ds(s * RPS + kk * 32, 32)])

        @pl.when(s == 0)
        def _():
            pltpu.sync_copy(zb.at[pl.ds(0, 8)], shared.at[pl.ds(HALF, 8)])

        def fire_idx(blk, b):
            flat0 = s * EPS2 + blk * B2
            pltpu.async_copy(src_hbm.at[pl.ds(flat0, B2)], gidx.at[b],
                             sem_i[b])
            pltpu.async_copy(dst_hbm.at[pl.ds(flat0, B2)], dbuf.at[b],
                             sem_i[b])
            pltpu.async_copy(dist_hbm.at[pl.ds(flat0, B2)], distb.at[b],
                             sem_i[b])

        def wait_idx(b):
            pltpu.make_async_copy(src_hbm.at[pl.ds(0, B2)], gidx.at[b],
                                  sem_i[b]).wait()
            pltpu.make_async_copy(dst_hbm.at[pl.ds(0, B2)], dbuf.at[b],
                                  sem_i[b]).wait()
            pltpu.make_async_copy(dist_hbm.at[pl.ds(0, B2)], distb.at[b],
                                  sem_i[b]).wait()

        def fire_gather(b):
            for j in range(NCH2):
                pltpu.async_copy(
                    hp_hbm.at[gidx.at[b, pl.ds(j * CH, CH)]],
                    rows.at[b, pl.ds(j * CH, CH)], sem_g[b])

        def wait_gather(b):
            for j in range(NCH2):
                pltpu.make_async_copy(hp_hbm.at[pl.ds(0, CH)],
                                      rows.at[b, pl.ds(j * CH, CH)],
                                      sem_g[b]).wait()

        def fire_scatter(b):
            for j in range(NCH2):
                pltpu.async_copy(rows.at[b, pl.ds(j * CH, CH)],
                                 shared.at[sidx.at[b, j]], sem_s[b],
                                 add=True)

        def wait_scatter(b):
            for j in range(NCH2):
                pltpu.make_async_copy(hp_hbm.at[pl.ds(0, CH)],
                                      rows.at[b, pl.ds(j * CH, CH)],
                                      sem_s[b]).wait()

        def compute(b):
            for p in range(B2 // L):
                dd = dbuf[b, pl.ds(p * L, L)]
                loc = dd - lo
                inr = (loc >= 0) & (loc < HALF)
                si = jnp.where(inr, loc, HALF)
                sidx[b, p // (CH // L), pl.ds((p % (CH // L)) * L, L)] = si

            @pl.loop(0, B2)
            def _(r):
                d = plsc.load_gather(
                    distb, [jnp.full((L,), b, jnp.int32),
                            jnp.full((L,), r, jnp.int32)])
                for q in range(4):
                    rows[b, r, pl.ds(q * L, L)] = (
                        rows[b, r, pl.ds(q * L, L)] * d)

        trash16 = jnp.full((L,), HALF, jnp.int32)
        for b in (0, 1):
            for j in range(NCH2):
                for o in range(CH // L):
                    sidx[b, j, pl.ds(o * L, L)] = trash16

        plsc.subcore_barrier()

        fire_scatter(1)
        fire_idx(0, 0)
        wait_idx(0)
        fire_gather(0)
        fire_idx(1, 1)

        @pl.loop(0, NBLK2 // 2)
        def _(ii):
            k = ii * 2
            for b in (0, 1):
                nxt = jnp.minimum(k + b + 2, NBLK2 - 1)
                wait_idx(1 - b)
                wait_scatter(1 - b)
                fire_gather(1 - b)
                wait_gather(b)
                compute(b)
                fire_idx(nxt, b)
                fire_scatter(b)

        wait_idx(1)
        wait_gather(0)
        wait_scatter(1)

        plsc.subcore_barrier()
        pltpu.sync_copy(shared.at[pl.ds(s * RPS, RPS)],
                        out_hbm.at[pl.ds(c * HALF + s * RPS, RPS)])

    return agg(hp, src1d, dst1d, dist1d)


def _node_body(hp_ref, ah_ref, deg_ref, w_ref, b_ref, g_ref, e_ref, o_ref):
    hp = hp_ref[...]
    ah = ah_ref[...]
    deg = deg_ref[:, 0:1]
    norm = jnp.where(deg > 0, 1.0 / jnp.maximum(deg, 1.0), 0.0)
    w = w_ref[...]
    y = _dot(hp, w[:64]) + _dot(ah * norm, w[64:]) + b_ref[...]
    y = _ln(y, g_ref[...], e_ref[...])
    o_ref[...] = jnp.maximum(y, 0.0)


def _node_mlp(hp, ahf, degf, l0W, l0b, l0g, l0be):
    blk = 1000
    full = lambda shape: pl.BlockSpec(shape, lambda i: (0, 0))
    return pl.pallas_call(
        _node_body,
        grid=(N // blk,),
        in_specs=[pl.BlockSpec((blk, 64), lambda i: (i, 0)),
                  pl.BlockSpec((blk, 64), lambda i: (i, 0)),
                  pl.BlockSpec((blk, L), lambda i: (i, 0)),
                  full((128, 64)), full((1, 64)), full((1, 64)), full((1, 64))],
        out_specs=pl.BlockSpec((blk, 64), lambda i: (i, 0)),
        out_shape=jax.ShapeDtypeStruct((N, 64), jnp.float32),
    )(hp, ahf, degf, l0W, l0b.reshape(1, 64), l0g.reshape(1, 64),
      l0be.reshape(1, 64))


def _sc_edge_product(hl, src1d, dst1d):
    mesh = plsc.VectorSubcoreMesh(core_axis_name="c", subcore_axis_name="s")

    @functools.partial(
        pl.kernel,
        out_type=jax.ShapeDtypeStruct((E_PAD, 64), jnp.float32),
        mesh=mesh,
        scratch_types=[
            pltpu.VMEM((2, B4), jnp.int32),
            pltpu.VMEM((2, B4), jnp.int32),
            pltpu.VMEM((2, B4, 64), jnp.float32),
            pltpu.VMEM((2, B4, 64), jnp.float32),
            pltpu.SemaphoreType.DMA, pltpu.SemaphoreType.DMA,
            pltpu.SemaphoreType.DMA, pltpu.SemaphoreType.DMA,
            pltpu.SemaphoreType.DMA, pltpu.SemaphoreType.DMA,
        ],
        compiler_params=_SC_PARAMS,
    )
    def prod(hl_hbm, src_hbm, dst_hbm, t_hbm, ui, vi, hu, hv,
             si0, si1, sg0, sg1, so0, so1):
        c = lax.axis_index("c")
        s = lax.axis_index("s")
        w = s * NC + c
        sem_i = [si0, si1]
        sem_g = [sg0, sg1]
        sem_o = [so0, so1]

        def fire_idx(blk, b):
            flat0 = w * EPW4 + blk * B4
            pltpu.async_copy(src_hbm.at[pl.ds(flat0, B4)], ui.at[b],
                             sem_i[b])
            pltpu.async_copy(dst_hbm.at[pl.ds(flat0, B4)], vi.at[b],
                             sem_i[b])

        def wait_idx(b):
            pltpu.make_async_copy(src_hbm.at[pl.ds(0, B4)], ui.at[b],
                                  sem_i[b]).wait()
            pltpu.make_async_copy(dst_hbm.at[pl.ds(0, B4)], vi.at[b],
                                  sem_i[b]).wait()

        def fire_gather(b):
            for j in range(NCH4):
                pltpu.async_copy(hl_hbm.at[ui.at[b, pl.ds(j * CH, CH)]],
                                 hu.at[b, pl.ds(j * CH, CH)], sem_g[b])
                pltpu.async_copy(hl_hbm.at[vi.at[b, pl.ds(j * CH, CH)]],
                                 hv.at[b, pl.ds(j * CH, CH)], sem_g[b])

        def wait_gather(b):
            for j in range(NCH4):
                pltpu.make_async_copy(hl_hbm.at[pl.ds(0, CH)],
                                      hu.at[b, pl.ds(j * CH, CH)],
                                      sem_g[b]).wait()
                pltpu.make_async_copy(hl_hbm.at[pl.ds(0, CH)],
                                      hv.at[b, pl.ds(j * CH, CH)],
                                      sem_g[b]).wait()

        def fire_out(blk, b):
            flat0 = w * EPW4 + blk * B4
            pltpu.async_copy(hu.at[b], t_hbm.at[pl.ds(flat0, B4)], sem_o[b])

        def wait_out(b):
            pltpu.make_async_copy(hl_hbm.at[pl.ds(0, B4)], hu.at[b],
                                  sem_o[b]).wait()

        def compute(b):
            @pl.loop(0, B4)
            def _(r):
                for q in range(4):
                    hu[b, r, pl.ds(q * L, L)] = (hu[b, r, pl.ds(q * L, L)]
                                                 * hv[b, r, pl.ds(q * L, L)])

        pltpu.async_copy(hu.at[1], t_hbm.at[pl.ds(E, B4)], sem_o[1])
        fire_idx(0, 0)
        wait_idx(0)
        fire_gather(0)
        fire_idx(1, 1)

        @pl.loop(0, NBLK4 // 2)
        def _(ii):
            k = ii * 2
            for b in (0, 1):
                nxt = jnp.minimum(k + b + 2, NBLK4 - 1)
                wait_idx(1 - b)
                wait_out(1 - b)
                fire_gather(1 - b)
                wait_gather(b)
                compute(b)
                fire_idx(nxt, b)
                fire_out(k + b, b)

        wait_idx(1)
        wait_gather(0)
        wait_out(1)

    return prod(hl, src1d, dst1d)


def _edge_body(t_ref, w1, b1, w2, b2, o_ref):
    y = jnp.maximum(_dot(t_ref[...], w1[...]) + b1[...], 0.0)
    o_ref[...] = _dot(y, w2[...]) + b2[...]


def _edge_mlp(t, eW1, eb1, eW2, eb2):
    blk = 3200
    full = lambda shape: pl.BlockSpec(shape, lambda i: (0, 0))
    return pl.pallas_call(
        _edge_body,
        grid=(E // blk,),
        in_specs=[pl.BlockSpec((blk, 64), lambda i: (i, 0)),
                  full((64, 32)), full((1, 32)), full((32, 2)), full((1, 2))],
        out_specs=pl.BlockSpec((blk, 2), lambda i: (i, 0)),
        out_shape=jax.ShapeDtypeStruct((E, 2), jnp.float32),
    )(t, eW1, eb1.reshape(1, 32), eW2, eb2.reshape(1, 2))


def kernel(h, edge_index, edge_dist, pW0, pb0, pg0, pbe0, pW1, pb1, pg1, pbe1,
           l0W, l0b, l0g, l0be, eW1, eb1, eW2, eb2):
    src = edge_index[0]
    dst = edge_index[1]
    npad = E_PAD - E
    src_p = jnp.concatenate([src, jnp.zeros((npad,), jnp.int32)])
    dst_p2 = jnp.concatenate([dst, jnp.full((npad,), N, jnp.int32)])
    dst_p4 = jnp.concatenate([dst, jnp.zeros((npad,), jnp.int32)])
    dist_p = jnp.concatenate([edge_dist, jnp.zeros((npad,), jnp.float32)])

    degf = _sc_degree(dst_p2)
    hp = _proj(h, pW0, pb0, pg0, pbe0, pW1, pb1, pg1, pbe1)
    ahf = _sc_aggregate(hp, src_p, dst_p2, dist_p)
    hl = _node_mlp(hp, ahf, degf, l0W, l0b, l0g, l0be)
    t = _sc_edge_product(hl, src_p, dst_p4)
    return _edge_mlp(t, eW1, eb1, eW2, eb2)


# speedup vs baseline: 2.2110x; 1.2153x over previous
import functools

import jax
import jax.numpy as jnp
from jax import lax
from jax.experimental import pallas as pl
from jax.experimental.pallas import tpu as pltpu
from jax.experimental.pallas import tpu_sc as plsc

N = 50000
E = 800000
NC = 2
NS = 16
L = 16

E_PAD = 819200
HALF = 25088
RPS = HALF // NS
SH_ROWS = HALF + 8
CH = 80
B2 = 160
NCH2 = B2 // CH
EPS2 = E_PAD // NS
NBLK2 = EPS2 // B2
BD = 1600
NCHD = BD // CH
NBLKD = EPS2 // BD
B4 = 400
NCH4 = B4 // CH
EPW4 = E_PAD // (NC * NS)
NBLK4 = EPW4 // B4

_HIGHEST = jax.lax.Precision.HIGHEST
_SC_PARAMS = pltpu.CompilerParams(needs_layout_passes=False,
                                  use_tc_tiling_on_sc=False)


def _dot(a, b, precision=_HIGHEST):
    return jax.lax.dot_general(a, b, (((1,), (0,)), ((), ())),
                               preferred_element_type=jnp.float32,
                               precision=precision)


def _ln(x, g, b, eps=1e-5):
    mu = jnp.mean(x, axis=-1, keepdims=True)
    var = jnp.mean((x - mu) ** 2, axis=-1, keepdims=True)
    return (x - mu) / jnp.sqrt(var + eps) * g + b


def _proj_body(h_ref, w0, b0, g0, e0, w1, b1, g1, e1, o_ref):
    h = h_ref[...]

    def chunk(x, W, b, g, be):
        y = _dot(x, W[...]) + b[...]
        y = _ln(y, g[...], be[...])
        return jnp.where(y >= 0, y, 0.01 * y)

    p0 = chunk(h[:, :64], w0, b0, g0, e0)
    p1 = chunk(h[:, 64:], w1, b1, g1, e1)
    o_ref[...] = jnp.concatenate([p0, p1], axis=1)


def _proj(h, pW0, pb0, pg0, pbe0, pW1, pb1, pg1, pbe1):
    blk = 1000
    full = lambda shape: pl.BlockSpec(shape, lambda i: (0, 0))
    return pl.pallas_call(
        _proj_body,
        grid=(N // blk,),
        in_specs=[pl.BlockSpec((blk, 128), lambda i: (i, 0)),
                  full((64, 32)), full((1, 32)), full((1, 32)), full((1, 32)),
                  full((64, 32)), full((1, 32)), full((1, 32)), full((1, 32))],
        out_specs=pl.BlockSpec((blk, 64), lambda i: (i, 0)),
        out_shape=jax.ShapeDtypeStruct((N, 64), jnp.float32),
    )(h, pW0, pb0.reshape(1, 32), pg0.reshape(1, 32), pbe0.reshape(1, 32),
      pW1, pb1.reshape(1, 32), pg1.reshape(1, 32), pbe1.reshape(1, 32))


DOUT = 50176
DRPS = DOUT // NS
EPSD = E_PAD // (2 * NS)
NBLKD2 = EPSD // BD


def _sc_degree(dst1d):
    mesh = plsc.VectorSubcoreMesh(core_axis_name="c", subcore_axis_name="s")

    @functools.partial(
        pl.kernel,
        out_type=jax.ShapeDtypeStruct((2, DOUT, L), jnp.float32),
        mesh=mesh,
        scratch_types=[
            pltpu.VMEM_SHARED((DOUT, L), jnp.float32),
            pltpu.VMEM((BD,), jnp.int32),
            pltpu.VMEM((NCHD, CH), jnp.int32),
            pltpu.VMEM((BD, L), jnp.float32),
            pltpu.VMEM((32, L), jnp.float32),
            pltpu.SemaphoreType.DMA,
        ],
        compiler_params=_SC_PARAMS,
    )
    def deg(dst_hbm, out_hbm, shared, dbuf, sidx, ones, zb, sem_s):
        c = lax.axis_index("c")
        s = lax.axis_index("s")

        zero16 = jnp.zeros((L,), jnp.float32)
        onev = (lax.iota(jnp.int32, L) == 0).astype(jnp.float32)

        @pl.loop(0, 32)
        def _(r):
            zb[r, pl.ds(0, L)] = zero16

        @pl.loop(0, BD)
        def _(r):
            ones[r, pl.ds(0, L)] = onev

        for kk in range(DRPS // 32):
            pltpu.sync_copy(zb, shared.at[pl.ds(s * DRPS + kk * 32, 32)])

        plsc.subcore_barrier()

        @pl.loop(0, NBLKD2)
        def _(i):
            flat0 = (c * NS + s) * EPSD + i * BD
            pltpu.sync_copy(dst_hbm.at[pl.ds(flat0, BD)], dbuf)
            for p in range(BD // L):
                dd = dbuf[pl.ds(p * L, L)]
                sidx[p // (CH // L), pl.ds((p % (CH // L)) * L, L)] = dd
            scs = [pltpu.async_copy(ones.at[pl.ds(j * CH, CH)],
                                    shared.at[sidx.at[j]], sem_s, add=True)
                   for j in range(NCHD)]
            for h_ in scs:
                h_.wait()

        plsc.subcore_barrier()
        pltpu.sync_copy(shared.at[pl.ds(s * DRPS, DRPS)],
                        out_hbm.at[c, pl.ds(s * DRPS, DRPS)])

    return deg(dst1d)


def _sc_aggregate(hp, src1d, dst1d, dist1d):
    mesh = plsc.VectorSubcoreMesh(core_axis_name="c", subcore_axis_name="s")

    @functools.partial(
        pl.kernel,
        out_type=jax.ShapeDtypeStruct((2 * HALF, 64), jnp.float32),
        mesh=mesh,
        scratch_types=[
            pltpu.VMEM_SHARED((SH_ROWS, 64), jnp.float32),
            pltpu.VMEM((2, B2), jnp.int32),
            pltpu.VMEM((2, NCH2, CH), jnp.int32),
            pltpu.VMEM((2, B2), jnp.int32),
            pltpu.VMEM((2, B2), jnp.float32),
            pltpu.VMEM((2, B2, 64), jnp.float32),
            pltpu.VMEM((32, 64), jnp.float32),
            pltpu.SemaphoreType.DMA, pltpu.SemaphoreType.DMA,
            pltpu.SemaphoreType.DMA, pltpu.SemaphoreType.DMA,
            pltpu.SemaphoreType.DMA, pltpu.SemaphoreType.DMA,
        ],
        compiler_params=_SC_PARAMS,
    )
    def agg(hp_hbm, src_hbm, dst_hbm, dist_hbm, out_hbm,
            shared, gidx, sidx, dbuf, distb, rows, zb,
            si0, si1, sg0, sg1, ss0, ss1):
        c = lax.axis_index("c")
        s = lax.axis_index("s")
        lo = c * HALF
        sem_i = [si0, si1]
        sem_g = [sg0, sg1]
        sem_s = [ss0, ss1]

        zero16 = jnp.zeros((L,), jnp.float32)

        @pl.loop(0, 32)
        def _(r):
            for q in range(4):
                zb[r, pl.ds(q * L, L)] = zero16

        for kk in range(49):
            pltpu.sync_copy(zb, shared.at[pl.ds(s * RPS + kk * 32, 32)])

        @pl.when(s == 0)
        def _():
            pltpu.sync_copy(zb.at[pl.ds(0, 8)], shared.at[pl.ds(HALF, 8)])

        def fire_idx(blk, b):
            flat0 = s * EPS2 + blk * B2
            pltpu.async_copy(src_hbm.at[pl.ds(flat0, B2)], gidx.at[b],
                             sem_i[b])
            pltpu.async_copy(dst_hbm.at[pl.ds(flat0, B2)], dbuf.at[b],
                             sem_i[b])
            pltpu.async_copy(dist_hbm.at[pl.ds(flat0, B2)], distb.at[b],
                             sem_i[b])

        def wait_idx(b):
            pltpu.make_async_copy(src_hbm.at[pl.ds(0, B2)], gidx.at[b],
                                  sem_i[b]).wait()
            pltpu.make_async_copy(dst_hbm.at[pl.ds(0, B2)], dbuf.at[b],
                                  sem_i[b]).wait()
            pltpu.make_async_copy(dist_hbm.at[pl.ds(0, B2)], distb.at[b],
                                  sem_i[b]).wait()

        def fire_gather(b):
            for j in range(NCH2):
                pltpu.async_copy(
                    hp_hbm.at[gidx.at[b, pl.ds(j * CH, CH)]],
                    rows.at[b, pl.ds(j * CH, CH)], sem_g[b])

        def wait_gather(b):
            for j in range(NCH2):
                pltpu.make_async_copy(hp_hbm.at[pl.ds(0, CH)],
                                      rows.at[b, pl.ds(j * CH, CH)],
                                      sem_g[b]).wait()

        def fire_scatter(b):
            for j in range(NCH2):
                pltpu.async_copy(rows.at[b, pl.ds(j * CH, CH)],
                                 shared.at[sidx.at[b, j]], sem_s[b],
                                 add=True)

        def wait_scatter(b):
            for j in range(NCH2):
                pltpu.make_async_copy(hp_hbm.at[pl.ds(0, CH)],
                                      rows.at[b, pl.ds(j * CH, CH)],
                                      sem_s[b]).wait()

        def compute(b):
            for p in range(B2 // L):
                dd = dbuf[b, pl.ds(p * L, L)]
                loc = dd - lo
                inr = (loc >= 0) & (loc < HALF)
                si = jnp.where(inr, loc, HALF)
                sidx[b, p // (CH // L), pl.ds((p % (CH // L)) * L, L)] = si

            @pl.loop(0, B2)
            def _(r):
                d = plsc.load_gather(
                    distb, [jnp.full((L,), b, jnp.int32),
                            jnp.full((L,), r, jnp.int32)])
                for q in range(4):
                    rows[b, r, pl.ds(q * L, L)] = (
                        rows[b, r, pl.ds(q * L, L)] * d)

        trash16 = jnp.full((L,), HALF, jnp.int32)
        for b in (0, 1):
            for j in range(NCH2):
                for o in range(CH // L):
                    sidx[b, j, pl.ds(o * L, L)] = trash16

        plsc.subcore_barrier()

        fire_scatter(1)
        fire_idx(0, 0)
        wait_idx(0)
        fire_gather(0)
        fire_idx(1, 1)

        @pl.loop(0, NBLK2 // 2)
        def _(ii):
            k = ii * 2
            for b in (0, 1):
                nxt = jnp.minimum(k + b + 2, NBLK2 - 1)
                wait_idx(1 - b)
                wait_scatter(1 - b)
                fire_gather(1 - b)
                wait_gather(b)
                compute(b)
                fire_idx(nxt, b)
                fire_scatter(b)

        wait_idx(1)
        wait_gather(0)
        wait_scatter(1)

        plsc.subcore_barrier()
        pltpu.sync_copy(shared.at[pl.ds(s * RPS, RPS)],
                        out_hbm.at[pl.ds(c * HALF + s * RPS, RPS)])

    return agg(hp, src1d, dst1d, dist1d)


def _node_body(hp_ref, ah_ref, d0_ref, d1_ref, w_ref, b_ref, g_ref, e_ref,
               o_ref):
    hp = hp_ref[...]
    ah = ah_ref[...]
    deg = d0_ref[0, :, 0:1] + d1_ref[0, :, 0:1]
    norm = jnp.where(deg > 0, 1.0 / jnp.maximum(deg, 1.0), 0.0)
    w = w_ref[...]
    y = _dot(hp, w[:64]) + _dot(ah * norm, w[64:]) + b_ref[...]
    y = _ln(y, g_ref[...], e_ref[...])
    o_ref[...] = jnp.maximum(y, 0.0)


def _node_mlp(hp, ahf, degf, l0W, l0b, l0g, l0be):
    blk = 1000
    full = lambda shape: pl.BlockSpec(shape, lambda i: (0, 0))
    return pl.pallas_call(
        _node_body,
        grid=(N // blk,),
        in_specs=[pl.BlockSpec((blk, 64), lambda i: (i, 0)),
                  pl.BlockSpec((blk, 64), lambda i: (i, 0)),
                  pl.BlockSpec((1, blk, L), lambda i: (0, i, 0)),
                  pl.BlockSpec((1, blk, L), lambda i: (1, i, 0)),
                  full((128, 64)), full((1, 64)), full((1, 64)), full((1, 64))],
        out_specs=pl.BlockSpec((blk, 64), lambda i: (i, 0)),
        out_shape=jax.ShapeDtypeStruct((N, 64), jnp.float32),
    )(hp, ahf, degf, degf, l0W, l0b.reshape(1, 64), l0g.reshape(1, 64),
      l0be.reshape(1, 64))


def _sc_edge_product(hl, src1d, dst1d):
    mesh = plsc.VectorSubcoreMesh(core_axis_name="c", subcore_axis_name="s")

    @functools.partial(
        pl.kernel,
        out_type=jax.ShapeDtypeStruct((E_PAD, 64), jnp.float32),
        mesh=mesh,
        scratch_types=[
            pltpu.VMEM((2, B4), jnp.int32),
            pltpu.VMEM((2, B4), jnp.int32),
            pltpu.VMEM((2, B4, 64), jnp.float32),
            pltpu.VMEM((2, B4, 64), jnp.float32),
            pltpu.SemaphoreType.DMA, pltpu.SemaphoreType.DMA,
            pltpu.SemaphoreType.DMA, pltpu.SemaphoreType.DMA,
            pltpu.SemaphoreType.DMA, pltpu.SemaphoreType.DMA,
        ],
        compiler_params=_SC_PARAMS,
    )
    def prod(hl_hbm, src_hbm, dst_hbm, t_hbm, ui, vi, hu, hv,
             si0, si1, sg0, sg1, so0, so1):
        c = lax.axis_index("c")
        s = lax.axis_index("s")
        w = s * NC + c
        sem_i = [si0, si1]
        sem_g = [sg0, sg1]
        sem_o = [so0, so1]

        def fire_idx(blk, b):
            flat0 = w * EPW4 + blk * B4
            pltpu.async_copy(src_hbm.at[pl.ds(flat0, B4)], ui.at[b],
                             sem_i[b])
            pltpu.async_copy(dst_hbm.at[pl.ds(flat0, B4)], vi.at[b],
                             sem_i[b])

        def wait_idx(b):
            pltpu.make_async_copy(src_hbm.at[pl.ds(0, B4)], ui.at[b],
                                  sem_i[b]).wait()
            pltpu.make_async_copy(dst_hbm.at[pl.ds(0, B4)], vi.at[b],
                                  sem_i[b]).wait()

        def fire_gather(b):
            for j in range(NCH4):
                pltpu.async_copy(hl_hbm.at[ui.at[b, pl.ds(j * CH, CH)]],
                                 hu.at[b, pl.ds(j * CH, CH)], sem_g[b])
                pltpu.async_copy(hl_hbm.at[vi.at[b, pl.ds(j * CH, CH)]],
                                 hv.at[b, pl.ds(j * CH, CH)], sem_g[b])

        def wait_gather(b):
            for j in range(NCH4):
                pltpu.make_async_copy(hl_hbm.at[pl.ds(0, CH)],
                                      hu.at[b, pl.ds(j * CH, CH)],
                                      sem_g[b]).wait()
                pltpu.make_async_copy(hl_hbm.at[pl.ds(0, CH)],
                                      hv.at[b, pl.ds(j * CH, CH)],
                                      sem_g[b]).wait()

        def fire_out(blk, b):
            flat0 = w * EPW4 + blk * B4
            pltpu.async_copy(hu.at[b], t_hbm.at[pl.ds(flat0, B4)], sem_o[b])

        def wait_out(b):
            pltpu.make_async_copy(hl_hbm.at[pl.ds(0, B4)], hu.at[b],
                                  sem_o[b]).wait()

        def compute(b):
            @pl.loop(0, B4)
            def _(r):
                for q in range(4):
                    hu[b, r, pl.ds(q * L, L)] = (hu[b, r, pl.ds(q * L, L)]
                                                 * hv[b, r, pl.ds(q * L, L)])

        pltpu.async_copy(hu.at[1], t_hbm.at[pl.ds(E + w * B4, B4)], sem_o[1])
        fire_idx(0, 0)
        wait_idx(0)
        fire_gather(0)
        fire_idx(1, 1)

        @pl.loop(0, NBLK4 // 2)
        def _(ii):
            k = ii * 2
            for b in (0, 1):
                nxt = jnp.minimum(k + b + 2, NBLK4 - 1)
                wait_idx(1 - b)
                wait_out(1 - b)
                fire_gather(1 - b)
                wait_gather(b)
                compute(b)
                fire_idx(nxt, b)
                fire_out(k + b, b)

        wait_idx(1)
        wait_gather(0)
        wait_out(1)

    return prod(hl, src1d, dst1d)


def _dot3(a, b):
    ah = a.astype(jnp.bfloat16)
    al = (a - ah.astype(jnp.float32)).astype(jnp.bfloat16)
    bh = b.astype(jnp.bfloat16)
    bl = (b - bh.astype(jnp.float32)).astype(jnp.bfloat16)
    return _dot(ah, bh, None) + (_dot(ah, bl, None) + _dot(al, bh, None))


def _edge_body(t_ref, w1, b1, w2, b2, o_ref):
    y = jnp.maximum(_dot3(t_ref[...], w1[...]) + b1[...], 0.0)
    o_ref[...] = _dot3(y, w2[...]) + b2[...]


def _edge_mlp(t, eW1, eb1, eW2, eb2):
    blk = 3200
    full = lambda shape: pl.BlockSpec(shape, lambda i: (0, 0))
    return pl.pallas_call(
        _edge_body,
        grid=(E // blk,),
        in_specs=[pl.BlockSpec((blk, 64), lambda i: (i, 0)),
                  full((64, 32)), full((1, 32)), full((32, 2)), full((1, 2))],
        out_specs=pl.BlockSpec((blk, 2), lambda i: (i, 0)),
        out_shape=jax.ShapeDtypeStruct((E, 2), jnp.float32),
    )(t, eW1, eb1.reshape(1, 32), eW2, eb2.reshape(1, 2))


def kernel(h, edge_index, edge_dist, pW0, pb0, pg0, pbe0, pW1, pb1, pg1, pbe1,
           l0W, l0b, l0g, l0be, eW1, eb1, eW2, eb2):
    src = edge_index[0]
    dst = edge_index[1]
    npad = E_PAD - E
    src_p = jnp.concatenate([src, jnp.zeros((npad,), jnp.int32)])
    dst_p2 = jnp.concatenate([dst, jnp.full((npad,), N, jnp.int32)])
    dst_p4 = jnp.concatenate([dst, jnp.zeros((npad,), jnp.int32)])
    dist_p = jnp.concatenate([edge_dist, jnp.zeros((npad,), jnp.float32)])

    degf = _sc_degree(dst_p2)
    hp = _proj(h, pW0, pb0, pg0, pbe0, pW1, pb1, pg1, pbe1)
    ahf = _sc_aggregate(hp, src_p, dst_p2, dist_p)
    hl = _node_mlp(hp, ahf, degf, l0W, l0b, l0g, l0be)
    t = _sc_edge_product(hl, src_p, dst_p4)
    return _edge_mlp(t, eW1, eb1, eW2, eb2)


# speedup vs baseline: 2.2117x; 1.0003x over previous
import functools

import jax
import jax.numpy as jnp
from jax import lax
from jax.experimental import pallas as pl
from jax.experimental.pallas import tpu as pltpu
from jax.experimental.pallas import tpu_sc as plsc

N = 50000
E = 800000
NC = 2
NS = 16
L = 16

E_PAD = 819200
HALF = 25088
RPS = HALF // NS
SH_ROWS = HALF + 8
CH = 80
B2 = 160
NCH2 = B2 // CH
EPS2 = E_PAD // NS
NBLK2 = EPS2 // B2
BD = 1600
NCHD = BD // CH
NBLKD = EPS2 // BD
B4 = 400
NCH4 = B4 // CH
EPW4 = E_PAD // (NC * NS)
NBLK4 = EPW4 // B4

_HIGHEST = jax.lax.Precision.HIGHEST
_SC_PARAMS = pltpu.CompilerParams(needs_layout_passes=False,
                                  use_tc_tiling_on_sc=False)


def _dot(a, b, precision=_HIGHEST):
    return jax.lax.dot_general(a, b, (((1,), (0,)), ((), ())),
                               preferred_element_type=jnp.float32,
                               precision=precision)


def _ln(x, g, b, eps=1e-5):
    mu = jnp.mean(x, axis=-1, keepdims=True)
    var = jnp.mean((x - mu) ** 2, axis=-1, keepdims=True)
    return (x - mu) / jnp.sqrt(var + eps) * g + b


def _proj_body(h_ref, w0, b0, g0, e0, w1, b1, g1, e1, o_ref):
    h = h_ref[...]

    def chunk(x, W, b, g, be):
        y = _dot(x, W[...]) + b[...]
        y = _ln(y, g[...], be[...])
        return jnp.where(y >= 0, y, 0.01 * y)

    p0 = chunk(h[:, :64], w0, b0, g0, e0)
    p1 = chunk(h[:, 64:], w1, b1, g1, e1)
    o_ref[...] = jnp.concatenate([p0, p1], axis=1)


def _proj(h, pW0, pb0, pg0, pbe0, pW1, pb1, pg1, pbe1):
    blk = 1000
    full = lambda shape: pl.BlockSpec(shape, lambda i: (0, 0))
    return pl.pallas_call(
        _proj_body,
        grid=(N // blk,),
        in_specs=[pl.BlockSpec((blk, 128), lambda i: (i, 0)),
                  full((64, 32)), full((1, 32)), full((1, 32)), full((1, 32)),
                  full((64, 32)), full((1, 32)), full((1, 32)), full((1, 32))],
        out_specs=pl.BlockSpec((blk, 64), lambda i: (i, 0)),
        out_shape=jax.ShapeDtypeStruct((N, 64), jnp.float32),
    )(h, pW0, pb0.reshape(1, 32), pg0.reshape(1, 32), pbe0.reshape(1, 32),
      pW1, pb1.reshape(1, 32), pg1.reshape(1, 32), pbe1.reshape(1, 32))


DOUT = 50176
DRPS = DOUT // NS
EPSD = E_PAD // (2 * NS)
NBLKD2 = EPSD // BD


def _sc_degree(dst1d):
    mesh = plsc.VectorSubcoreMesh(core_axis_name="c", subcore_axis_name="s")

    @functools.partial(
        pl.kernel,
        out_type=jax.ShapeDtypeStruct((2, DOUT, L), jnp.float32),
        mesh=mesh,
        scratch_types=[
            pltpu.VMEM_SHARED((DOUT, L), jnp.float32),
            pltpu.VMEM((BD,), jnp.int32),
            pltpu.VMEM((NCHD, CH), jnp.int32),
            pltpu.VMEM((BD, L), jnp.float32),
            pltpu.VMEM((32, L), jnp.float32),
            pltpu.SemaphoreType.DMA,
        ],
        compiler_params=_SC_PARAMS,
    )
    def deg(dst_hbm, out_hbm, shared, dbuf, sidx, ones, zb, sem_s):
        c = lax.axis_index("c")
        s = lax.axis_index("s")

        zero16 = jnp.zeros((L,), jnp.float32)
        onev = (lax.iota(jnp.int32, L) == 0).astype(jnp.float32)

        @pl.loop(0, 32)
        def _(r):
            zb[r, pl.ds(0, L)] = zero16

        @pl.loop(0, BD)
        def _(r):
            ones[r, pl.ds(0, L)] = onev

        for kk in range(DRPS // 32):
            pltpu.sync_copy(zb, shared.at[pl.ds(s * DRPS + kk * 32, 32)])

        plsc.subcore_barrier()

        @pl.loop(0, NBLKD2)
        def _(i):
            flat0 = (c * NS + s) * EPSD + i * BD
            pltpu.sync_copy(dst_hbm.at[pl.ds(flat0, BD)], dbuf)
            for p in range(BD // L):
                dd = dbuf[pl.ds(p * L, L)]
                sidx[p // (CH // L), pl.ds((p % (CH // L)) * L, L)] = dd
            scs = [pltpu.async_copy(ones.at[pl.ds(j * CH, CH)],
                                    shared.at[sidx.at[j]], sem_s, add=True)
                   for j in range(NCHD)]
            for h_ in scs:
                h_.wait()

        plsc.subcore_barrier()
        pltpu.sync_copy(shared.at[pl.ds(s * DRPS, DRPS)],
                        out_hbm.at[c, pl.ds(s * DRPS, DRPS)])

    return deg(dst1d)


def _sc_aggregate(hp, src1d, dst1d, dist1d):
    mesh = plsc.VectorSubcoreMesh(core_axis_name="c", subcore_axis_name="s")

    @functools.partial(
        pl.kernel,
        out_type=jax.ShapeDtypeStruct((2 * HALF, 64), jnp.float32),
        mesh=mesh,
        scratch_types=[
            pltpu.VMEM_SHARED((SH_ROWS, 64), jnp.float32),
            pltpu.VMEM((2, B2), jnp.int32),
            pltpu.VMEM((2, NCH2, CH), jnp.int32),
            pltpu.VMEM((2, B2), jnp.int32),
            pltpu.VMEM((2, B2), jnp.float32),
            pltpu.VMEM((2, B2, 64), jnp.float32),
            pltpu.VMEM((32, 64), jnp.float32),
            pltpu.SemaphoreType.DMA, pltpu.SemaphoreType.DMA,
            pltpu.SemaphoreType.DMA, pltpu.SemaphoreType.DMA,
            pltpu.SemaphoreType.DMA, pltpu.SemaphoreType.DMA,
        ],
        compiler_params=_SC_PARAMS,
    )
    def agg(hp_hbm, src_hbm, dst_hbm, dist_hbm, out_hbm,
            shared, gidx, sidx, dbuf, distb, rows, zb,
            si0, si1, sg0, sg1, ss0, ss1):
        c = lax.axis_index("c")
        s = lax.axis_index("s")
        lo = c * HALF
        sem_i = [si0, si1]
        sem_g = [sg0, sg1]
        sem_s = [ss0, ss1]

        zero16 = jnp.zeros((L,), jnp.float32)

        @pl.loop(0, 32)
        def _(r):
            for q in range(4):
                zb[r, pl.ds(q * L, L)] = zero16

        for kk in range(49):
            pltpu.sync_copy(zb, shared.at[pl.ds(s * RPS + kk * 32, 32)])

        @pl.when(s == 0)
        def _():
            pltpu.sync_copy(zb.at[pl.ds(0, 8)], shared.at[pl.ds(HALF, 8)])

        def fire_idx(blk, b):
            flat0 = s * EPS2 + blk * B2
            pltpu.async_copy(src_hbm.at[pl.ds(flat0, B2)], gidx.at[b],
                             sem_i[b])
            pltpu.async_copy(dst_hbm.at[pl.ds(flat0, B2)], dbuf.at[b],
                             sem_i[b])
            pltpu.async_copy(dist_hbm.at[pl.ds(flat0, B2)], distb.at[b],
                             sem_i[b])

        def wait_idx(b):
            pltpu.make_async_copy(src_hbm.at[pl.ds(0, B2)], gidx.at[b],
                                  sem_i[b]).wait()
            pltpu.make_async_copy(dst_hbm.at[pl.ds(0, B2)], dbuf.at[b],
                                  sem_i[b]).wait()
            pltpu.make_async_copy(dist_hbm.at[pl.ds(0, B2)], distb.at[b],
                                  sem_i[b]).wait()

        def fire_gather(b):
            for j in range(NCH2):
                pltpu.async_copy(
                    hp_hbm.at[gidx.at[b, pl.ds(j * CH, CH)]],
                    rows.at[b, pl.ds(j * CH, CH)], sem_g[b])

        def wait_gather(b):
            for j in range(NCH2):
                pltpu.make_async_copy(hp_hbm.at[pl.ds(0, CH)],
                                      rows.at[b, pl.ds(j * CH, CH)],
                                      sem_g[b]).wait()

        def fire_scatter(b):
            for j in range(NCH2):
                pltpu.async_copy(rows.at[b, pl.ds(j * CH, CH)],
                                 shared.at[sidx.at[b, j]], sem_s[b],
                                 add=True)

        def wait_scatter(b):
            for j in range(NCH2):
                pltpu.make_async_copy(hp_hbm.at[pl.ds(0, CH)],
                                      rows.at[b, pl.ds(j * CH, CH)],
                                      sem_s[b]).wait()

        def compute(b):
            for p in range(B2 // L):
                dd = dbuf[b, pl.ds(p * L, L)]
                loc = dd - lo
                inr = (loc >= 0) & (loc < HALF)
                si = jnp.where(inr, loc, HALF)
                sidx[b, p // (CH // L), pl.ds((p % (CH // L)) * L, L)] = si

            @pl.loop(0, B2)
            def _(r):
                d = plsc.load_gather(
                    distb, [jnp.full((L,), b, jnp.int32),
                            jnp.full((L,), r, jnp.int32)])
                for q in range(4):
                    rows[b, r, pl.ds(q * L, L)] = (
                        rows[b, r, pl.ds(q * L, L)] * d)

        trash16 = jnp.full((L,), HALF, jnp.int32)
        for b in (0, 1):
            for j in range(NCH2):
                for o in range(CH // L):
                    sidx[b, j, pl.ds(o * L, L)] = trash16

        plsc.subcore_barrier()

        fire_scatter(1)
        fire_idx(0, 0)
        wait_idx(0)
        fire_gather(0)
        fire_idx(1, 1)

        @pl.loop(0, NBLK2 // 2)
        def _(ii):
            k = ii * 2
            for b in (0, 1):
                nxt = jnp.minimum(k + b + 2, NBLK2 - 1)
                wait_idx(1 - b)
                wait_scatter(1 - b)
                fire_gather(1 - b)
                wait_gather(b)
                compute(b)
                fire_idx(nxt, b)
                fire_scatter(b)

        wait_idx(1)
        wait_gather(0)
        wait_scatter(1)

        plsc.subcore_barrier()
        pltpu.sync_copy(shared.at[pl.ds(s * RPS, RPS)],
                        out_hbm.at[pl.ds(c * HALF + s * RPS, RPS)])

    return agg(hp, src1d, dst1d, dist1d)


def _node_body(hp_ref, ah_ref, d0_ref, d1_ref, w_ref, b_ref, g_ref, e_ref,
               o_ref):
    hp = hp_ref[...]
    ah = ah_ref[...]
    deg = d0_ref[0, :, 0:1] + d1_ref[0, :, 0:1]
    norm = jnp.where(deg > 0, 1.0 / jnp.maximum(deg, 1.0), 0.0)
    w = w_ref[...]
    y = _dot(hp, w[:64]) + _dot(ah * norm, w[64:]) + b_ref[...]
    y = _ln(y, g_ref[...], e_ref[...])
    o_ref[...] = jnp.maximum(y, 0.0)


def _node_mlp(hp, ahf, degf, l0W, l0b, l0g, l0be):
    blk = 1000
    full = lambda shape: pl.BlockSpec(shape, lambda i: (0, 0))
    return pl.pallas_call(
        _node_body,
        grid=(N // blk,),
        in_specs=[pl.BlockSpec((blk, 64), lambda i: (i, 0)),
                  pl.BlockSpec((blk, 64), lambda i: (i, 0)),
                  pl.BlockSpec((1, blk, L), lambda i: (0, i, 0)),
                  pl.BlockSpec((1, blk, L), lambda i: (1, i, 0)),
                  full((128, 64)), full((1, 64)), full((1, 64)), full((1, 64))],
        out_specs=pl.BlockSpec((blk, 64), lambda i: (i, 0)),
        out_shape=jax.ShapeDtypeStruct((N, 64), jnp.float32),
    )(hp, ahf, degf, degf, l0W, l0b.reshape(1, 64), l0g.reshape(1, 64),
      l0be.reshape(1, 64))


def _sc_edge_product(hl, src1d, dst1d):
    mesh = plsc.VectorSubcoreMesh(core_axis_name="c", subcore_axis_name="s")

    @functools.partial(
        pl.kernel,
        out_type=jax.ShapeDtypeStruct((E_PAD, 64), jnp.float32),
        mesh=mesh,
        scratch_types=[
            pltpu.VMEM((2, B4), jnp.int32),
            pltpu.VMEM((2, B4), jnp.int32),
            pltpu.VMEM((2, B4, 64), jnp.float32),
            pltpu.VMEM((2, B4, 64), jnp.float32),
            pltpu.SemaphoreType.DMA, pltpu.SemaphoreType.DMA,
            pltpu.SemaphoreType.DMA, pltpu.SemaphoreType.DMA,
            pltpu.SemaphoreType.DMA, pltpu.SemaphoreType.DMA,
        ],
        compiler_params=_SC_PARAMS,
    )
    def prod(hl_hbm, src_hbm, dst_hbm, t_hbm, ui, vi, hu, hv,
             si0, si1, sg0, sg1, so0, so1):
        c = lax.axis_index("c")
        s = lax.axis_index("s")
        w = c * NS + s
        sem_i = [si0, si1]
        sem_g = [sg0, sg1]
        sem_o = [so0, so1]

        def fire_idx(blk, b):
            flat0 = w * EPW4 + blk * B4
            pltpu.async_copy(src_hbm.at[pl.ds(flat0, B4)], ui.at[b],
                             sem_i[b])
            pltpu.async_copy(dst_hbm.at[pl.ds(flat0, B4)], vi.at[b],
                             sem_i[b])

        def wait_idx(b):
            pltpu.make_async_copy(src_hbm.at[pl.ds(0, B4)], ui.at[b],
                                  sem_i[b]).wait()
            pltpu.make_async_copy(dst_hbm.at[pl.ds(0, B4)], vi.at[b],
                                  sem_i[b]).wait()

        def fire_gather(b):
            for j in range(NCH4):
                pltpu.async_copy(hl_hbm.at[ui.at[b, pl.ds(j * CH, CH)]],
                                 hu.at[b, pl.ds(j * CH, CH)], sem_g[b])
                pltpu.async_copy(hl_hbm.at[vi.at[b, pl.ds(j * CH, CH)]],
                                 hv.at[b, pl.ds(j * CH, CH)], sem_g[b])

        def wait_gather(b):
            for j in range(NCH4):
                pltpu.make_async_copy(hl_hbm.at[pl.ds(0, CH)],
                                      hu.at[b, pl.ds(j * CH, CH)],
                                      sem_g[b]).wait()
                pltpu.make_async_copy(hl_hbm.at[pl.ds(0, CH)],
                                      hv.at[b, pl.ds(j * CH, CH)],
                                      sem_g[b]).wait()

        def fire_out(blk, b):
            flat0 = w * EPW4 + blk * B4
            pltpu.async_copy(hu.at[b], t_hbm.at[pl.ds(flat0, B4)], sem_o[b])

        def wait_out(b):
            pltpu.make_async_copy(hl_hbm.at[pl.ds(0, B4)], hu.at[b],
                                  sem_o[b]).wait()

        def compute(b):
            @pl.loop(0, B4)
            def _(r):
                for q in range(4):
                    hu[b, r, pl.ds(q * L, L)] = (hu[b, r, pl.ds(q * L, L)]
                                                 * hv[b, r, pl.ds(q * L, L)])

        pltpu.async_copy(hu.at[1], t_hbm.at[pl.ds(E + w * B4, B4)], sem_o[1])
        fire_idx(0, 0)
        wait_idx(0)
        fire_gather(0)
        fire_idx(1, 1)

        @pl.loop(0, NBLK4 // 2)
        def _(ii):
            k = ii * 2
            for b in (0, 1):
                nxt = jnp.minimum(k + b + 2, NBLK4 - 1)
                wait_idx(1 - b)
                wait_out(1 - b)
                fire_gather(1 - b)
                wait_gather(b)
                compute(b)
                fire_idx(nxt, b)
                fire_out(k + b, b)

        wait_idx(1)
        wait_gather(0)
        wait_out(1)

    return prod(hl, src1d, dst1d)


def _dot3(a, b):
    ah = a.astype(jnp.bfloat16)
    al = (a - ah.astype(jnp.float32)).astype(jnp.bfloat16)
    bh = b.astype(jnp.bfloat16)
    bl = (b - bh.astype(jnp.float32)).astype(jnp.bfloat16)
    return _dot(ah, bh, None) + (_dot(ah, bl, None) + _dot(al, bh, None))


def _edge_body(t_ref, w1, b1, w2, b2, o_ref):
    y = jnp.maximum(_dot3(t_ref[...], w1[...]) + b1[...], 0.0)
    o_ref[...] = _dot3(y, w2[...]) + b2[...]


def _edge_mlp(t, eW1, eb1, eW2, eb2):
    blk = 3200
    full = lambda shape: pl.BlockSpec(shape, lambda i: (0, 0))
    return pl.pallas_call(
        _edge_body,
        grid=(E // blk,),
        in_specs=[pl.BlockSpec((blk, 64), lambda i: (i, 0)),
                  full((64, 32)), full((1, 32)), full((32, 2)), full((1, 2))],
        out_specs=pl.BlockSpec((blk, 2), lambda i: (i, 0)),
        out_shape=jax.ShapeDtypeStruct((E, 2), jnp.float32),
    )(t, eW1, eb1.reshape(1, 32), eW2, eb2.reshape(1, 2))


def kernel(h, edge_index, edge_dist, pW0, pb0, pg0, pbe0, pW1, pb1, pg1, pbe1,
           l0W, l0b, l0g, l0be, eW1, eb1, eW2, eb2):
    src = edge_index[0]
    dst = edge_index[1]
    npad = E_PAD - E
    src_p = jnp.concatenate([src, jnp.zeros((npad,), jnp.int32)])
    dst_p2 = jnp.concatenate([dst, jnp.full((npad,), N, jnp.int32)])
    dst_p4 = jnp.concatenate([dst, jnp.zeros((npad,), jnp.int32)])
    dist_p = jnp.concatenate([edge_dist, jnp.zeros((npad,), jnp.float32)])

    degf = _sc_degree(dst_p2)
    hp = _proj(h, pW0, pb0, pg0, pbe0, pW1, pb1, pg1, pbe1)
    ahf = _sc_aggregate(hp, src_p, dst_p2, dist_p)
    hl = _node_mlp(hp, ahf, degf, l0W, l0b, l0g, l0be)
    t = _sc_edge_product(hl, src_p, dst_p4)
    return _edge_mlp(t, eW1, eb1, eW2, eb2)


# speedup vs baseline: 2.6188x; 1.1841x over previous
import functools

import jax
import jax.numpy as jnp
from jax import lax
from jax.experimental import pallas as pl
from jax.experimental.pallas import tpu as pltpu
from jax.experimental.pallas import tpu_sc as plsc

N = 50000
E = 800000
NC = 2
NS = 16
L = 16

E_PAD = 819200
CH = 80
B2 = 160
NCH2 = B2 // CH
EPS2 = E_PAD // NS
NBLK2 = EPS2 // B2
BD = 1600
NCHD = BD // CH
NBLKD = EPS2 // BD
B4 = 400
NCH4 = B4 // CH
EPW4 = E_PAD // (NC * NS)
NBLK4 = EPW4 // B4

_HIGHEST = jax.lax.Precision.HIGHEST
_SC_PARAMS = pltpu.CompilerParams(needs_layout_passes=False,
                                  use_tc_tiling_on_sc=False)


def _dot(a, b, precision=_HIGHEST):
    return jax.lax.dot_general(a, b, (((1,), (0,)), ((), ())),
                               preferred_element_type=jnp.float32,
                               precision=precision)


def _ln(x, g, b, eps=1e-5):
    mu = jnp.mean(x, axis=-1, keepdims=True)
    var = jnp.mean((x - mu) ** 2, axis=-1, keepdims=True)
    return (x - mu) / jnp.sqrt(var + eps) * g + b


def _proj_body(h_ref, w0, b0, g0, e0, w1, b1, g1, e1, o_ref):
    h = h_ref[...]

    def chunk(x, W, b, g, be):
        y = _dot(x, W[...]) + b[...]
        y = _ln(y, g[...], be[...])
        return jnp.where(y >= 0, y, 0.01 * y)

    o_ref[0] = chunk(h[:, :64], w0, b0, g0, e0)
    o_ref[1] = chunk(h[:, 64:], w1, b1, g1, e1)


def _proj(h, pW0, pb0, pg0, pbe0, pW1, pb1, pg1, pbe1):
    blk = 1000
    full = lambda shape: pl.BlockSpec(shape, lambda i: (0, 0))
    return pl.pallas_call(
        _proj_body,
        grid=(N // blk,),
        in_specs=[pl.BlockSpec((blk, 128), lambda i: (i, 0)),
                  full((64, 32)), full((1, 32)), full((1, 32)), full((1, 32)),
                  full((64, 32)), full((1, 32)), full((1, 32)), full((1, 32))],
        out_specs=pl.BlockSpec((2, blk, 32), lambda i: (0, i, 0)),
        out_shape=jax.ShapeDtypeStruct((2, N, 32), jnp.float32),
    )(h, pW0, pb0.reshape(1, 32), pg0.reshape(1, 32), pbe0.reshape(1, 32),
      pW1, pb1.reshape(1, 32), pg1.reshape(1, 32), pbe1.reshape(1, 32))


DOUT = 50176
DRPS = DOUT // NS
EPSD = E_PAD // (2 * NS)
NBLKD2 = EPSD // BD


def _sc_degree(dst1d):
    mesh = plsc.VectorSubcoreMesh(core_axis_name="c", subcore_axis_name="s")

    @functools.partial(
        pl.kernel,
        out_type=jax.ShapeDtypeStruct((2, DOUT, L), jnp.float32),
        mesh=mesh,
        scratch_types=[
            pltpu.VMEM_SHARED((DOUT, L), jnp.float32),
            pltpu.VMEM((BD,), jnp.int32),
            pltpu.VMEM((NCHD, CH), jnp.int32),
            pltpu.VMEM((BD, L), jnp.float32),
            pltpu.VMEM((32, L), jnp.float32),
            pltpu.SemaphoreType.DMA,
        ],
        compiler_params=_SC_PARAMS,
    )
    def deg(dst_hbm, out_hbm, shared, dbuf, sidx, ones, zb, sem_s):
        c = lax.axis_index("c")
        s = lax.axis_index("s")

        zero16 = jnp.zeros((L,), jnp.float32)
        onev = (lax.iota(jnp.int32, L) == 0).astype(jnp.float32)

        @pl.loop(0, 32)
        def _(r):
            zb[r, pl.ds(0, L)] = zero16

        @pl.loop(0, BD)
        def _(r):
            ones[r, pl.ds(0, L)] = onev

        for kk in range(DRPS // 32):
            pltpu.sync_copy(zb, shared.at[pl.ds(s * DRPS + kk * 32, 32)])

        plsc.subcore_barrier()

        @pl.loop(0, NBLKD2)
        def _(i):
            flat0 = (c * NS + s) * EPSD + i * BD
            pltpu.sync_copy(dst_hbm.at[pl.ds(flat0, BD)], dbuf)
            for p in range(BD // L):
                dd = dbuf[pl.ds(p * L, L)]
                sidx[p // (CH // L), pl.ds((p % (CH // L)) * L, L)] = dd
            scs = [pltpu.async_copy(ones.at[pl.ds(j * CH, CH)],
                                    shared.at[sidx.at[j]], sem_s, add=True)
                   for j in range(NCHD)]
            for h_ in scs:
                h_.wait()

        plsc.subcore_barrier()
        pltpu.sync_copy(shared.at[pl.ds(s * DRPS, DRPS)],
                        out_hbm.at[c, pl.ds(s * DRPS, DRPS)])

    return deg(dst1d)


JUNK = DOUT - 8


def _sc_aggregate(hp, src1d, dst1d, dist1d):
    mesh = plsc.VectorSubcoreMesh(core_axis_name="c", subcore_axis_name="s")

    @functools.partial(
        pl.kernel,
        out_type=jax.ShapeDtypeStruct((2, DOUT, 32), jnp.float32),
        mesh=mesh,
        scratch_types=[
            pltpu.VMEM_SHARED((DOUT, 32), jnp.float32),
            pltpu.VMEM((2, B2), jnp.int32),
            pltpu.VMEM((2, NCH2, CH), jnp.int32),
            pltpu.VMEM((2, B2), jnp.int32),
            pltpu.VMEM((2, B2), jnp.float32),
            pltpu.VMEM((2, B2, 32), jnp.float32),
            pltpu.VMEM((32, 32), jnp.float32),
            pltpu.SemaphoreType.DMA, pltpu.SemaphoreType.DMA,
            pltpu.SemaphoreType.DMA, pltpu.SemaphoreType.DMA,
            pltpu.SemaphoreType.DMA, pltpu.SemaphoreType.DMA,
        ],
        compiler_params=_SC_PARAMS,
    )
    def agg(hp_hbm, src_hbm, dst_hbm, dist_hbm, out_hbm,
            shared, gidx, sidx, dbuf, distb, rows, zb,
            si0, si1, sg0, sg1, ss0, ss1):
        c = lax.axis_index("c")
        s = lax.axis_index("s")
        sem_i = [si0, si1]
        sem_g = [sg0, sg1]
        sem_s = [ss0, ss1]

        zero16 = jnp.zeros((L,), jnp.float32)

        @pl.loop(0, 32)
        def _(r):
            for q in range(2):
                zb[r, pl.ds(q * L, L)] = zero16

        for kk in range(DRPS // 32):
            pltpu.sync_copy(zb, shared.at[pl.ds(s * DRPS + kk * 32, 32)])

        def fire_idx(blk, b):
            flat0 = s * EPS2 + blk * B2
            pltpu.async_copy(src_hbm.at[pl.ds(flat0, B2)], gidx.at[b],
                             sem_i[b])
            pltpu.async_copy(dst_hbm.at[pl.ds(flat0, B2)], dbuf.at[b],
                             sem_i[b])
            pltpu.async_copy(dist_hbm.at[pl.ds(flat0, B2)], distb.at[b],
                             sem_i[b])

        def wait_idx(b):
            pltpu.make_async_copy(src_hbm.at[pl.ds(0, B2)], gidx.at[b],
                                  sem_i[b]).wait()
            pltpu.make_async_copy(dst_hbm.at[pl.ds(0, B2)], dbuf.at[b],
                                  sem_i[b]).wait()
            pltpu.make_async_copy(dist_hbm.at[pl.ds(0, B2)], distb.at[b],
                                  sem_i[b]).wait()

        def adjust(b):
            off = c * N
            for p in range(B2 // L):
                gidx[b, pl.ds(p * L, L)] = gidx[b, pl.ds(p * L, L)] + off

        def fire_gather(b):
            for j in range(NCH2):
                pltpu.async_copy(
                    hp_hbm.at[gidx.at[b, pl.ds(j * CH, CH)]],
                    rows.at[b, pl.ds(j * CH, CH)], sem_g[b])

        def wait_gather(b):
            for j in range(NCH2):
                pltpu.make_async_copy(hp_hbm.at[pl.ds(0, CH)],
                                      rows.at[b, pl.ds(j * CH, CH)],
                                      sem_g[b]).wait()

        def fire_scatter(b):
            for j in range(NCH2):
                pltpu.async_copy(rows.at[b, pl.ds(j * CH, CH)],
                                 shared.at[sidx.at[b, j]], sem_s[b],
                                 add=True)

        def wait_scatter(b):
            for j in range(NCH2):
                pltpu.make_async_copy(hp_hbm.at[pl.ds(0, CH)],
                                      rows.at[b, pl.ds(j * CH, CH)],
                                      sem_s[b]).wait()

        def compute(b):
            for p in range(B2 // L):
                sidx[b, p // (CH // L), pl.ds((p % (CH // L)) * L, L)] = (
                    dbuf[b, pl.ds(p * L, L)])

            @pl.loop(0, B2)
            def _(r):
                d = plsc.load_gather(
                    distb, [jnp.full((L,), b, jnp.int32),
                            jnp.full((L,), r, jnp.int32)])
                for q in range(2):
                    rows[b, r, pl.ds(q * L, L)] = (
                        rows[b, r, pl.ds(q * L, L)] * d)

        junk16 = jnp.full((L,), JUNK, jnp.int32)
        for b in (0, 1):
            for j in range(NCH2):
                for o in range(CH // L):
                    sidx[b, j, pl.ds(o * L, L)] = junk16

        plsc.subcore_barrier()

        fire_scatter(1)
        fire_idx(0, 0)
        wait_idx(0)
        adjust(0)
        fire_gather(0)
        fire_idx(1, 1)

        @pl.loop(0, NBLK2 // 2)
        def _(ii):
            k = ii * 2
            for b in (0, 1):
                nxt = jnp.minimum(k + b + 2, NBLK2 - 1)
                wait_idx(1 - b)
                adjust(1 - b)
                wait_scatter(1 - b)
                fire_gather(1 - b)
                wait_gather(b)
                compute(b)
                fire_idx(nxt, b)
                fire_scatter(b)

        wait_idx(1)
        wait_gather(0)
        wait_scatter(1)

        plsc.subcore_barrier()
        pltpu.sync_copy(shared.at[pl.ds(s * DRPS, DRPS)],
                        out_hbm.at[c, pl.ds(s * DRPS, DRPS)])

    return agg(hp.reshape(2 * N, 32), src1d, dst1d, dist1d)


def _node_body(hp_ref, a0_ref, a1_ref, d0_ref, d1_ref, w_ref, b_ref, g_ref,
               e_ref, o_ref):
    hp = jnp.concatenate([hp_ref[0], hp_ref[1]], axis=1)
    ah = jnp.concatenate([a0_ref[0], a1_ref[0]], axis=1)
    deg = d0_ref[0, :, 0:1] + d1_ref[0, :, 0:1]
    norm = jnp.where(deg > 0, 1.0 / jnp.maximum(deg, 1.0), 0.0)
    w = w_ref[...]
    y = _dot(hp, w[:64]) + _dot(ah * norm, w[64:]) + b_ref[...]
    y = _ln(y, g_ref[...], e_ref[...])
    o_ref[...] = jnp.maximum(y, 0.0)


def _node_mlp(hp, ahf, degf, l0W, l0b, l0g, l0be):
    blk = 1000
    full = lambda shape: pl.BlockSpec(shape, lambda i: (0, 0))
    return pl.pallas_call(
        _node_body,
        grid=(N // blk,),
        in_specs=[pl.BlockSpec((2, blk, 32), lambda i: (0, i, 0)),
                  pl.BlockSpec((1, blk, 32), lambda i: (0, i, 0)),
                  pl.BlockSpec((1, blk, 32), lambda i: (1, i, 0)),
                  pl.BlockSpec((1, blk, L), lambda i: (0, i, 0)),
                  pl.BlockSpec((1, blk, L), lambda i: (1, i, 0)),
                  full((128, 64)), full((1, 64)), full((1, 64)), full((1, 64))],
        out_specs=pl.BlockSpec((blk, 64), lambda i: (i, 0)),
        out_shape=jax.ShapeDtypeStruct((N, 64), jnp.float32),
    )(hp, ahf, ahf, degf, degf, l0W, l0b.reshape(1, 64), l0g.reshape(1, 64),
      l0be.reshape(1, 64))


def _sc_edge_product(hl, src1d, dst1d):
    mesh = plsc.VectorSubcoreMesh(core_axis_name="c", subcore_axis_name="s")

    @functools.partial(
        pl.kernel,
        out_type=jax.ShapeDtypeStruct((E_PAD, 64), jnp.float32),
        mesh=mesh,
        scratch_types=[
            pltpu.VMEM((2, B4), jnp.int32),
            pltpu.VMEM((2, B4), jnp.int32),
            pltpu.VMEM((2, B4, 64), jnp.float32),
            pltpu.VMEM((2, B4, 64), jnp.float32),
            pltpu.SemaphoreType.DMA, pltpu.SemaphoreType.DMA,
            pltpu.SemaphoreType.DMA, pltpu.SemaphoreType.DMA,
            pltpu.SemaphoreType.DMA, pltpu.SemaphoreType.DMA,
        ],
        compiler_params=_SC_PARAMS,
    )
    def prod(hl_hbm, src_hbm, dst_hbm, t_hbm, ui, vi, hu, hv,
             si0, si1, sg0, sg1, so0, so1):
        c = lax.axis_index("c")
        s = lax.axis_index("s")
        w = c * NS + s
        sem_i = [si0, si1]
        sem_g = [sg0, sg1]
        sem_o = [so0, so1]

        def fire_idx(blk, b):
            flat0 = w * EPW4 + blk * B4
            pltpu.async_copy(src_hbm.at[pl.ds(flat0, B4)], ui.at[b],
                             sem_i[b])
            pltpu.async_copy(dst_hbm.at[pl.ds(flat0, B4)], vi.at[b],
                             sem_i[b])

        def wait_idx(b):
            pltpu.make_async_copy(src_hbm.at[pl.ds(0, B4)], ui.at[b],
                                  sem_i[b]).wait()
            pltpu.make_async_copy(dst_hbm.at[pl.ds(0, B4)], vi.at[b],
                                  sem_i[b]).wait()

        def fire_gather(b):
            for j in range(NCH4):
                pltpu.async_copy(hl_hbm.at[ui.at[b, pl.ds(j * CH, CH)]],
                                 hu.at[b, pl.ds(j * CH, CH)], sem_g[b])
                pltpu.async_copy(hl_hbm.at[vi.at[b, pl.ds(j * CH, CH)]],
                                 hv.at[b, pl.ds(j * CH, CH)], sem_g[b])

        def wait_gather(b):
            for j in range(NCH4):
                pltpu.make_async_copy(hl_hbm.at[pl.ds(0, CH)],
                                      hu.at[b, pl.ds(j * CH, CH)],
                                      sem_g[b]).wait()
                pltpu.make_async_copy(hl_hbm.at[pl.ds(0, CH)],
                                      hv.at[b, pl.ds(j * CH, CH)],
                                      sem_g[b]).wait()

        def fire_out(blk, b):
            flat0 = w * EPW4 + blk * B4
            pltpu.async_copy(hu.at[b], t_hbm.at[pl.ds(flat0, B4)], sem_o[b])

        def wait_out(b):
            pltpu.make_async_copy(hl_hbm.at[pl.ds(0, B4)], hu.at[b],
                                  sem_o[b]).wait()

        def compute(b):
            @pl.loop(0, B4)
            def _(r):
                for q in range(4):
                    hu[b, r, pl.ds(q * L, L)] = (hu[b, r, pl.ds(q * L, L)]
                                                 * hv[b, r, pl.ds(q * L, L)])

        pltpu.async_copy(hu.at[1], t_hbm.at[pl.ds(E + w * B4, B4)], sem_o[1])
        fire_idx(0, 0)
        wait_idx(0)
        fire_gather(0)
        fire_idx(1, 1)

        @pl.loop(0, NBLK4 // 2)
        def _(ii):
            k = ii * 2
            for b in (0, 1):
                nxt = jnp.minimum(k + b + 2, NBLK4 - 1)
                wait_idx(1 - b)
                wait_out(1 - b)
                fire_gather(1 - b)
                wait_gather(b)
                compute(b)
                fire_idx(nxt, b)
                fire_out(k + b, b)

        wait_idx(1)
        wait_gather(0)
        wait_out(1)

    return prod(hl, src1d, dst1d)


def _dot3(a, b):
    ah = a.astype(jnp.bfloat16)
    al = (a - ah.astype(jnp.float32)).astype(jnp.bfloat16)
    bh = b.astype(jnp.bfloat16)
    bl = (b - bh.astype(jnp.float32)).astype(jnp.bfloat16)
    return _dot(ah, bh, None) + (_dot(ah, bl, None) + _dot(al, bh, None))


def _edge_body(t_ref, w1, b1, w2, b2, o_ref):
    y = jnp.maximum(_dot3(t_ref[...], w1[...]) + b1[...], 0.0)
    o_ref[...] = _dot3(y, w2[...]) + b2[...]


def _edge_mlp(t, eW1, eb1, eW2, eb2):
    blk = 3200
    full = lambda shape: pl.BlockSpec(shape, lambda i: (0, 0))
    return pl.pallas_call(
        _edge_body,
        grid=(E // blk,),
        in_specs=[pl.BlockSpec((blk, 64), lambda i: (i, 0)),
                  full((64, 32)), full((1, 32)), full((32, 2)), full((1, 2))],
        out_specs=pl.BlockSpec((blk, 2), lambda i: (i, 0)),
        out_shape=jax.ShapeDtypeStruct((E, 2), jnp.float32),
    )(t, eW1, eb1.reshape(1, 32), eW2, eb2.reshape(1, 2))


def kernel(h, edge_index, edge_dist, pW0, pb0, pg0, pbe0, pW1, pb1, pg1, pbe1,
           l0W, l0b, l0g, l0be, eW1, eb1, eW2, eb2):
    src = edge_index[0]
    dst = edge_index[1]
    npad = E_PAD - E
    src_p = jnp.concatenate([src, jnp.zeros((npad,), jnp.int32)])
    dst_p2 = jnp.concatenate([dst, jnp.full((npad,), N, jnp.int32)])
    dst_p4 = jnp.concatenate([dst, jnp.zeros((npad,), jnp.int32)])
    dist_p = jnp.concatenate([edge_dist, jnp.zeros((npad,), jnp.float32)])

    degf = _sc_degree(dst_p2)
    hp = _proj(h, pW0, pb0, pg0, pbe0, pW1, pb1, pg1, pbe1)
    ahf = _sc_aggregate(hp, src_p, dst_p2, dist_p)
    hl = _node_mlp(hp, ahf, degf, l0W, l0b, l0g, l0be)
    t = _sc_edge_product(hl, src_p, dst_p4)
    return _edge_mlp(t, eW1, eb1, eW2, eb2)
